# baseline (speedup 1.0000x reference)
import functools

import jax
import jax.numpy as jnp
from jax.experimental import pallas as pl
from jax.experimental.pallas import tpu as pltpu


def _mil_body(inv_ref, x_ref, p_ref, n_ref, out_ref):
    b = pl.program_id(0)
    x = x_ref[0]
    m = (p_ref[0] > 0.5) & (n_ref[0] > 0.5)
    probs = jax.nn.sigmoid(x)
    s = jnp.sum(jnp.where(m, probs, 0.0))
    c = jnp.sum(m.astype(jnp.float32))
    p = s / c
    inv = inv_ref[b]
    term = -inv * jnp.log(p) - (1.0 - inv) * jnp.log(1.0 - p)

    @pl.when(b == 0)
    def _():
        out_ref[...] = jnp.zeros_like(out_ref)

    out_ref[...] = out_ref[...] + term


def kernel(cancer_logits, prostate_mask, needle_mask, involvement, grade_group):
    B, _, H, W = cancer_logits.shape
    x = cancer_logits.reshape(B, H, W)
    pm = prostate_mask.reshape(B, H, W)
    nm = needle_mask.reshape(B, H, W)
    inv = involvement

    img_spec = pl.BlockSpec((1, H, W), lambda b: (b, 0, 0))
    out = pl.pallas_call(
        _mil_body,
        grid=(B,),
        in_specs=[
            pl.BlockSpec(memory_space=pltpu.SMEM),
            img_spec,
            img_spec,
            img_spec,
        ],
        out_specs=pl.BlockSpec((1, 1), lambda b: (0, 0)),
        out_shape=jax.ShapeDtypeStruct((1, 1), jnp.float32),
    )(inv, x, pm, nm)
    return out[0, 0]
